# baseline (speedup 1.0000x reference)
import functools

import jax
import jax.numpy as jnp
from jax import lax
from jax.experimental import pallas as pl
from jax.experimental.pallas import tpu as pltpu
from jax.experimental.pallas import tpu_sc as plsc

N = 10000
NP = 10240
E = 320000
D = 128
DE = 16

NC = 2
NS = 16
NW = NC * NS
EPW = E // NW
CK = 80
NCHUNK = EPW // CK
RPS = NP // NS

_mesh = plsc.VectorSubcoreMesh(
    core_axis_name="c", subcore_axis_name="s", num_cores=NC, num_subcores=NS)


@functools.partial(
    pl.kernel,
    out_type=jax.ShapeDtypeStruct((NC, NP, D), jnp.float32),
    mesh=_mesh,
    scratch_types=[
        pltpu.VMEM((NCHUNK, CK), jnp.int32),
        pltpu.VMEM((CK, D), jnp.float32),
        pltpu.VMEM_SHARED((NP, D), jnp.float32),
        pltpu.SemaphoreType.DMA,
    ],
)
def _sc_degree(dst_hbm, ones_hbm, zeros_hbm, out_hbm, idx_v, ones_v, acc_sh,
               sem):
    c = lax.axis_index("c")
    s = lax.axis_index("s")
    wid = c * NS + s
    pltpu.sync_copy(zeros_hbm.at[pl.ds(s * RPS, RPS)],
                    acc_sh.at[pl.ds(s * RPS, RPS)])
    pltpu.sync_copy(dst_hbm.at[wid], idx_v)
    pltpu.sync_copy(ones_hbm, ones_v)
    plsc.subcore_barrier()

    def fire(m, carry):
        pltpu.async_copy(ones_v, acc_sh.at[idx_v.at[m]], sem, add=True)
        return carry

    def drain(m, carry):
        pltpu.make_async_copy(ones_v, acc_sh.at[idx_v.at[m]], sem).wait()
        return carry

    lax.fori_loop(0, NCHUNK, fire, 0)
    lax.fori_loop(0, NCHUNK, drain, 0)
    plsc.subcore_barrier()
    pltpu.sync_copy(acc_sh.at[pl.ds(s * RPS, RPS)],
                    out_hbm.at[c, pl.ds(s * RPS, RPS)])


@functools.partial(
    pl.kernel,
    out_type=jax.ShapeDtypeStruct((NC, NP, D), jnp.float32),
    mesh=_mesh,
    scratch_types=[
        pltpu.VMEM((EPW,), jnp.int32),
        pltpu.VMEM((NCHUNK, CK), jnp.int32),
        pltpu.VMEM((2, CK, D), jnp.float32),
        pltpu.VMEM_SHARED((NP, D), jnp.float32),
        pltpu.SemaphoreType.DMA((2,)),
    ],
)
def _sc_aggregate(hs_hbm, src_hbm, dst_hbm, zeros_hbm, out_hbm,
                  sidx_v, didx_v, rows_v, acc_sh, sg):
    c = lax.axis_index("c")
    s = lax.axis_index("s")
    wid = c * NS + s
    pltpu.sync_copy(zeros_hbm.at[pl.ds(s * RPS, RPS)],
                    acc_sh.at[pl.ds(s * RPS, RPS)])
    pltpu.sync_copy(src_hbm.at[wid], sidx_v)
    pltpu.sync_copy(dst_hbm.at[wid], didx_v)
    plsc.subcore_barrier()

    def issue_g(m, k):
        pltpu.async_copy(hs_hbm.at[sidx_v.at[pl.ds(m * CK, CK)]],
                         rows_v.at[k], sg.at[k])

    def wait_g(m, k):
        pltpu.make_async_copy(hs_hbm.at[sidx_v.at[pl.ds(m * CK, CK)]],
                              rows_v.at[k], sg.at[k]).wait()

    for k in range(2):
        issue_g(k, k)

    def body(mm, carry):
        for k in range(2):
            cc = 2 * mm + k
            wait_g(cc, k)
            pltpu.sync_copy(rows_v.at[k], acc_sh.at[didx_v.at[cc]], add=True)

            @pl.when(cc + 2 < NCHUNK)
            def _(cc=cc, k=k):
                issue_g(cc + 2, k)
        return carry

    lax.fori_loop(0, NCHUNK // 2, body, 0)
    for cc in range(2 * (NCHUNK // 2), NCHUNK):
        k = cc % 2
        wait_g(cc, k)
        pltpu.sync_copy(rows_v.at[k], acc_sh.at[didx_v.at[cc]], add=True)
    plsc.subcore_barrier()
    pltpu.sync_copy(acc_sh.at[pl.ds(s * RPS, RPS)],
                    out_hbm.at[c, pl.ds(s * RPS, RPS)])


@functools.partial(
    pl.kernel,
    out_type=jax.ShapeDtypeStruct((E,), jnp.float32),
    mesh=_mesh,
    compiler_params=pltpu.CompilerParams(needs_layout_passes=False),
    scratch_types=[
        pltpu.VMEM((NCHUNK, CK), jnp.int32),
        pltpu.VMEM((NCHUNK, CK), jnp.int32),
        pltpu.VMEM((2, CK, D), jnp.float32),
        pltpu.VMEM((EPW,), jnp.float32),
        pltpu.VMEM((D,), jnp.float32),
        pltpu.VMEM((16,), jnp.float32),
        pltpu.SemaphoreType.DMA((2,)),
        pltpu.SemaphoreType.DMA((2,)),
    ],
)
def _sc_edge_mlp(a_hbm, b_hbm, c_hbm, row_hbm, col_hbm, w2_hbm,
                 bm2_hbm, out_hbm, ridx_v, cidx_v, av, ov, w2v, bm2v,
                 sa, sadd):
    c = lax.axis_index("c")
    s = lax.axis_index("s")
    wid = c * NS + s
    ebase = pl.multiple_of(wid * EPW, 8)
    pltpu.sync_copy(row_hbm.at[wid], ridx_v)
    pltpu.sync_copy(col_hbm.at[wid], cidx_v)
    pltpu.sync_copy(w2_hbm, w2v)
    pltpu.sync_copy(bm2_hbm, bm2v)
    w = [w2v[pl.ds(16 * d, 16)] for d in range(D // 16)]
    bmv = bm2v[...]
    lanes = lax.iota(jnp.int32, 16)
    half = lanes < 8

    def initov(i, carry):
        ov[pl.ds(i * 16, 16)] = bmv
        return carry

    lax.fori_loop(0, EPW // 16, initov, 0)

    def issue_a(m, k):
        off = pl.multiple_of(ebase + m * CK, 8)
        pltpu.async_copy(c_hbm.at[pl.ds(off, CK)], av.at[k], sa.at[k])

    def wait_a(m, k):
        off = pl.multiple_of(ebase + m * CK, 8)
        pltpu.make_async_copy(c_hbm.at[pl.ds(off, CK)], av.at[k],
                              sa.at[k]).wait()

    def issue_adds(m, k):
        pltpu.async_copy(a_hbm.at[ridx_v.at[m]], av.at[k], sadd.at[k],
                         add=True)
        pltpu.async_copy(b_hbm.at[cidx_v.at[m]], av.at[k], sadd.at[k],
                         add=True)

    def wait_adds(m, k):
        for _ in range(2):
            pltpu.make_async_copy(b_hbm.at[cidx_v.at[m]], av.at[k],
                                  sadd.at[k]).wait()

    def compute(m, k):
        def group(g, icarry):
            for l in range(16):
                j = g * 16 + l
                acc = jnp.zeros((16,), jnp.float32)
                for d in range(D // 16):
                    sl = pl.ds(16 * d, 16)
                    t = av[k, j, sl]
                    acc = acc + jnp.maximum(t, 0.0) * w[d]
                folded = acc + lax.rev(acc, (0,))
                eidx = jnp.full((16,), m * CK + j, jnp.int32)
                plsc.addupdate_scatter(ov, [eidx], folded, mask=half)
            return icarry

        lax.fori_loop(0, CK // 16, group, 0)

    issue_a(0, 0)
    wait_a(0, 0)
    issue_adds(0, 0)
    issue_a(1, 1)

    def body(mm, carry):
        for k in range(2):
            m = 2 * mm + k
            nxt = m + 1

            @pl.when(nxt < NCHUNK)
            def _(nxt=nxt, k=k):
                wait_a(nxt, 1 - k)
                issue_adds(nxt, 1 - k)

            @pl.when(m < NCHUNK)
            def _(m=m, k=k):
                wait_adds(m, k)
                compute(m, k)

            @pl.when(nxt + 1 < NCHUNK)
            def _(nxt=nxt, k=k):
                issue_a(nxt + 1, k)
        return carry

    lax.fori_loop(0, (NCHUNK + 1) // 2, body, 0)
    pltpu.sync_copy(ov, out_hbm.at[pl.ds(ebase, EPW)])


_BM = 640
_BE = 2000


def _tc_hs0_body(x_ref, w_ref, d0_ref, d1_ref, hs_ref, dinv_ref):
    deg = d0_ref[:, 0:1] + d1_ref[:, 0:1] + 1.0
    dinv = lax.rsqrt(deg)
    h = jnp.dot(x_ref[...], w_ref[...], preferred_element_type=jnp.float32)
    hs_ref[...] = h * dinv
    dinv_ref[...] = dinv


def _tc_hs0(x, w1, degp0, degp1):
    return pl.pallas_call(
        _tc_hs0_body,
        grid=(NP // _BM,),
        in_specs=[
            pl.BlockSpec((_BM, D), lambda i: (i, 0)),
            pl.BlockSpec((D, D), lambda i: (0, 0)),
            pl.BlockSpec((_BM, D), lambda i: (i, 0)),
            pl.BlockSpec((_BM, D), lambda i: (i, 0)),
        ],
        out_specs=[
            pl.BlockSpec((_BM, D), lambda i: (i, 0)),
            pl.BlockSpec((_BM, 1), lambda i: (i, 0)),
        ],
        out_shape=[
            jax.ShapeDtypeStruct((NP, D), jnp.float32),
            jax.ShapeDtypeStruct((NP, 1), jnp.float32),
        ],
    )(x, w1, degp0, degp1)


def _tc_layer_body(a0_ref, a1_ref, hs_ref, dinv_ref, b_ref, w_ref, out_ref):
    dinv = dinv_ref[...]
    h = jnp.maximum(
        (a0_ref[...] + a1_ref[...] + hs_ref[...]) * dinv + b_ref[...], 0.0)
    out_ref[...] = (
        jnp.dot(h, w_ref[...], preferred_element_type=jnp.float32) * dinv)


def _tc_layer(acc0, acc1, hs, dinv, b, w):
    return pl.pallas_call(
        _tc_layer_body,
        grid=(NP // _BM,),
        in_specs=[
            pl.BlockSpec((_BM, D), lambda i: (i, 0)),
            pl.BlockSpec((_BM, D), lambda i: (i, 0)),
            pl.BlockSpec((_BM, D), lambda i: (i, 0)),
            pl.BlockSpec((_BM, 1), lambda i: (i, 0)),
            pl.BlockSpec((1, D), lambda i: (0, 0)),
            pl.BlockSpec((D, D), lambda i: (0, 0)),
        ],
        out_specs=pl.BlockSpec((_BM, D), lambda i: (i, 0)),
        out_shape=jax.ShapeDtypeStruct((NP, D), jnp.float32),
    )(acc0, acc1, hs, dinv, b, w)


def _tc_node_ab_body(a0_ref, a1_ref, hs_ref, dinv_ref, b_ref, wr_ref, wc_ref,
                     aout_ref, bout_ref):
    h = jnp.maximum(
        (a0_ref[...] + a1_ref[...] + hs_ref[...]) * dinv_ref[...] + b_ref[...],
        0.0)
    aout_ref[...] = jnp.dot(h, wr_ref[...], preferred_element_type=jnp.float32)
    bout_ref[...] = jnp.dot(h, wc_ref[...], preferred_element_type=jnp.float32)


def _tc_node_ab(acc0, acc1, hs, dinv, b, wr, wc):
    return pl.pallas_call(
        _tc_node_ab_body,
        grid=(NP // _BM,),
        in_specs=[
            pl.BlockSpec((_BM, D), lambda i: (i, 0)),
            pl.BlockSpec((_BM, D), lambda i: (i, 0)),
            pl.BlockSpec((_BM, D), lambda i: (i, 0)),
            pl.BlockSpec((_BM, 1), lambda i: (i, 0)),
            pl.BlockSpec((1, D), lambda i: (0, 0)),
            pl.BlockSpec((D, D), lambda i: (0, 0)),
            pl.BlockSpec((D, D), lambda i: (0, 0)),
        ],
        out_specs=[
            pl.BlockSpec((_BM, D), lambda i: (i, 0)),
            pl.BlockSpec((_BM, D), lambda i: (i, 0)),
        ],
        out_shape=[
            jax.ShapeDtypeStruct((NP, D), jnp.float32),
            jax.ShapeDtypeStruct((NP, D), jnp.float32),
        ],
    )(acc0, acc1, hs, dinv, b, wr, wc)


def _tc_edge_c_body(ea_ref, w_ref, b_ref, out_ref):
    out_ref[...] = (
        jnp.dot(ea_ref[...], w_ref[...], preferred_element_type=jnp.float32)
        + b_ref[...])


def _tc_edge_c(edge_attr, we, bm1):
    return pl.pallas_call(
        _tc_edge_c_body,
        grid=(E // _BE,),
        in_specs=[
            pl.BlockSpec((_BE, DE), lambda i: (i, 0)),
            pl.BlockSpec((DE, D), lambda i: (0, 0)),
            pl.BlockSpec((1, D), lambda i: (0, 0)),
        ],
        out_specs=pl.BlockSpec((_BE, D), lambda i: (i, 0)),
        out_shape=jax.ShapeDtypeStruct((E, D), jnp.float32),
    )(edge_attr, we, bm1)


def kernel(x, edge_index, edge_attr, W1, b1, W2, b2, Wm1, bm1, Wm2, bm2):
    src2 = edge_index[0].astype(jnp.int32).reshape(NW, EPW)
    src = src2.reshape(NW, NCHUNK, CK)
    dst = edge_index[1].astype(jnp.int32).reshape(NW, NCHUNK, CK)
    xp = jnp.pad(x, ((0, NP - N), (0, 0)))
    zeros128 = jnp.zeros((NP, D), jnp.float32)
    ones128 = jnp.ones((CK, D), jnp.float32)

    degp = _sc_degree(dst, ones128, zeros128)
    hs0, dinv = _tc_hs0(xp, W1, degp[0], degp[1])
    accp1 = _sc_aggregate(hs0, src2, dst, zeros128)
    hs1 = _tc_layer(accp1[0], accp1[1], hs0, dinv, b1.reshape(1, D), W2)
    accp2 = _sc_aggregate(hs1, src2, dst, zeros128)
    a_t, b_t = _tc_node_ab(accp2[0], accp2[1], hs1, dinv, b2.reshape(1, D),
                           Wm1[:D], Wm1[D:2 * D])
    c_t = _tc_edge_c(edge_attr, Wm1[2 * D:], bm1.reshape(1, D))
    logits = _sc_edge_mlp(a_t, b_t, c_t, src, dst, Wm2[:, 0],
                          jnp.full((16,), bm2[0], jnp.float32))
    return logits


# speedup vs baseline: 1.0774x; 1.0774x over previous
import functools

import jax
import jax.numpy as jnp
from jax import lax
from jax.experimental import pallas as pl
from jax.experimental.pallas import tpu as pltpu
from jax.experimental.pallas import tpu_sc as plsc

N = 10000
NP = 10240
E = 320000
D = 128
DE = 16

NC = 2
NS = 16
NW = NC * NS
EPW = E // NW
CK = 80
NCHUNK = EPW // CK
RPS = NP // NS

_mesh = plsc.VectorSubcoreMesh(
    core_axis_name="c", subcore_axis_name="s", num_cores=NC, num_subcores=NS)


@functools.partial(
    pl.kernel,
    out_type=jax.ShapeDtypeStruct((NC, NP, D), jnp.float32),
    mesh=_mesh,
    scratch_types=[
        pltpu.VMEM((NCHUNK, CK), jnp.int32),
        pltpu.VMEM((CK, D), jnp.float32),
        pltpu.VMEM_SHARED((NP, D), jnp.float32),
        pltpu.SemaphoreType.DMA,
    ],
)
def _sc_degree(dst_hbm, ones_hbm, zeros_hbm, out_hbm, idx_v, ones_v, acc_sh,
               sem):
    c = lax.axis_index("c")
    s = lax.axis_index("s")
    wid = c * NS + s
    pltpu.sync_copy(zeros_hbm.at[pl.ds(s * RPS, RPS)],
                    acc_sh.at[pl.ds(s * RPS, RPS)])
    pltpu.sync_copy(dst_hbm.at[wid], idx_v)
    pltpu.sync_copy(ones_hbm, ones_v)
    plsc.subcore_barrier()

    def fire(m, carry):
        pltpu.async_copy(ones_v, acc_sh.at[idx_v.at[m]], sem, add=True)
        return carry

    def drain(m, carry):
        pltpu.make_async_copy(ones_v, acc_sh.at[idx_v.at[m]], sem).wait()
        return carry

    lax.fori_loop(0, NCHUNK, fire, 0)
    lax.fori_loop(0, NCHUNK, drain, 0)
    plsc.subcore_barrier()
    pltpu.sync_copy(acc_sh.at[pl.ds(s * RPS, RPS)],
                    out_hbm.at[c, pl.ds(s * RPS, RPS)])


@functools.partial(
    pl.kernel,
    out_type=jax.ShapeDtypeStruct((NC, NP, D), jnp.float32),
    mesh=_mesh,
    scratch_types=[
        pltpu.VMEM((EPW,), jnp.int32),
        pltpu.VMEM((NCHUNK, CK), jnp.int32),
        pltpu.VMEM((2, CK, D), jnp.float32),
        pltpu.VMEM_SHARED((NP, D), jnp.float32),
        pltpu.SemaphoreType.DMA((2,)),
    ],
)
def _sc_aggregate(hs_hbm, src_hbm, dst_hbm, zeros_hbm, out_hbm,
                  sidx_v, didx_v, rows_v, acc_sh, sg):
    c = lax.axis_index("c")
    s = lax.axis_index("s")
    wid = c * NS + s
    pltpu.sync_copy(zeros_hbm.at[pl.ds(s * RPS, RPS)],
                    acc_sh.at[pl.ds(s * RPS, RPS)])
    pltpu.sync_copy(src_hbm.at[wid], sidx_v)
    pltpu.sync_copy(dst_hbm.at[wid], didx_v)
    plsc.subcore_barrier()

    def issue_g(m, k):
        pltpu.async_copy(hs_hbm.at[sidx_v.at[pl.ds(m * CK, CK)]],
                         rows_v.at[k], sg.at[k])

    def wait_g(m, k):
        pltpu.make_async_copy(hs_hbm.at[sidx_v.at[pl.ds(m * CK, CK)]],
                              rows_v.at[k], sg.at[k]).wait()

    for k in range(2):
        issue_g(k, k)

    def body(mm, carry):
        for k in range(2):
            cc = 2 * mm + k
            wait_g(cc, k)
            pltpu.sync_copy(rows_v.at[k], acc_sh.at[didx_v.at[cc]], add=True)

            @pl.when(cc + 2 < NCHUNK)
            def _(cc=cc, k=k):
                issue_g(cc + 2, k)
        return carry

    lax.fori_loop(0, NCHUNK // 2, body, 0)
    for cc in range(2 * (NCHUNK // 2), NCHUNK):
        k = cc % 2
        wait_g(cc, k)
        pltpu.sync_copy(rows_v.at[k], acc_sh.at[didx_v.at[cc]], add=True)
    plsc.subcore_barrier()
    pltpu.sync_copy(acc_sh.at[pl.ds(s * RPS, RPS)],
                    out_hbm.at[c, pl.ds(s * RPS, RPS)])


@functools.partial(
    pl.kernel,
    out_type=jax.ShapeDtypeStruct((E,), jnp.float32),
    mesh=_mesh,
    compiler_params=pltpu.CompilerParams(needs_layout_passes=False),
    scratch_types=[
        pltpu.VMEM((NCHUNK, CK), jnp.int32),
        pltpu.VMEM((NCHUNK, CK), jnp.int32),
        pltpu.VMEM((2, CK, D), jnp.float32),
        pltpu.VMEM((2, CK, D), jnp.float32),
        pltpu.VMEM((EPW,), jnp.float32),
        pltpu.VMEM((D,), jnp.float32),
        pltpu.VMEM((16,), jnp.float32),
        pltpu.SemaphoreType.DMA((2,)),
        pltpu.SemaphoreType.DMA((2,)),
        pltpu.SemaphoreType.DMA((2,)),
    ],
)
def _sc_edge_mlp(a_hbm, b_hbm, c_hbm, row_hbm, col_hbm, w2_hbm,
                 bm2_hbm, out_hbm, ridx_v, cidx_v, av, bv, ov, w2v, bm2v,
                 sa, sadd, sb):
    c = lax.axis_index("c")
    s = lax.axis_index("s")
    wid = c * NS + s
    ebase = pl.multiple_of(wid * EPW, 8)
    pltpu.sync_copy(row_hbm.at[wid], ridx_v)
    pltpu.sync_copy(col_hbm.at[wid], cidx_v)
    pltpu.sync_copy(w2_hbm, w2v)
    pltpu.sync_copy(bm2_hbm, bm2v)
    w = [w2v[pl.ds(16 * d, 16)] for d in range(D // 16)]
    bmv = bm2v[...]
    lanes = lax.iota(jnp.int32, 16)

    def issue_a(m, k):
        off = pl.multiple_of(ebase + m * CK, 8)
        pltpu.async_copy(c_hbm.at[pl.ds(off, CK)], av.at[k], sa.at[k])
        pltpu.async_copy(b_hbm.at[cidx_v.at[m]], bv.at[k], sb.at[k])

    def wait_a(m, k):
        off = pl.multiple_of(ebase + m * CK, 8)
        pltpu.make_async_copy(c_hbm.at[pl.ds(off, CK)], av.at[k],
                              sa.at[k]).wait()

    def issue_adds(m, k):
        pltpu.async_copy(a_hbm.at[ridx_v.at[m]], av.at[k], sadd.at[k],
                         add=True)

    def wait_adds(m, k):
        pltpu.make_async_copy(a_hbm.at[ridx_v.at[m]], av.at[k],
                              sadd.at[k]).wait()
        pltpu.make_async_copy(b_hbm.at[cidx_v.at[m]], bv.at[k],
                              sb.at[k]).wait()

    def compute(m, k):
        def group(g, icarry):
            res = bmv
            for l in range(16):
                j = g * 16 + l
                acc = jnp.zeros((16,), jnp.float32)
                for d in range(D // 16):
                    sl = pl.ds(16 * d, 16)
                    t = av[k, j, sl] + bv[k, j, sl]
                    acc = acc + jnp.maximum(t, 0.0) * w[d]
                s = jnp.sum(acc)
                res = jnp.where(lanes == l, res + s, res)
            ov[pl.ds(m * CK + g * 16, 16)] = res
            return icarry

        lax.fori_loop(0, CK // 16, group, 0)

    issue_a(0, 0)
    wait_a(0, 0)
    issue_adds(0, 0)
    issue_a(1, 1)

    def body(mm, carry):
        for k in range(2):
            m = 2 * mm + k
            nxt = m + 1

            @pl.when(nxt < NCHUNK)
            def _(nxt=nxt, k=k):
                wait_a(nxt, 1 - k)
                issue_adds(nxt, 1 - k)

            @pl.when(m < NCHUNK)
            def _(m=m, k=k):
                wait_adds(m, k)
                compute(m, k)

            @pl.when(nxt + 1 < NCHUNK)
            def _(nxt=nxt, k=k):
                issue_a(nxt + 1, k)
        return carry

    lax.fori_loop(0, (NCHUNK + 1) // 2, body, 0)
    pltpu.sync_copy(ov, out_hbm.at[pl.ds(ebase, EPW)])


_BM = 640
_BE = 2000


def _tc_hs0_body(x_ref, w_ref, d0_ref, d1_ref, hs_ref, dinv_ref):
    deg = d0_ref[:, 0:1] + d1_ref[:, 0:1] + 1.0
    dinv = lax.rsqrt(deg)
    h = jnp.dot(x_ref[...], w_ref[...], preferred_element_type=jnp.float32)
    hs_ref[...] = h * dinv
    dinv_ref[...] = dinv


def _tc_hs0(x, w1, degp0, degp1):
    return pl.pallas_call(
        _tc_hs0_body,
        grid=(NP // _BM,),
        in_specs=[
            pl.BlockSpec((_BM, D), lambda i: (i, 0)),
            pl.BlockSpec((D, D), lambda i: (0, 0)),
            pl.BlockSpec((_BM, D), lambda i: (i, 0)),
            pl.BlockSpec((_BM, D), lambda i: (i, 0)),
        ],
        out_specs=[
            pl.BlockSpec((_BM, D), lambda i: (i, 0)),
            pl.BlockSpec((_BM, 1), lambda i: (i, 0)),
        ],
        out_shape=[
            jax.ShapeDtypeStruct((NP, D), jnp.float32),
            jax.ShapeDtypeStruct((NP, 1), jnp.float32),
        ],
    )(x, w1, degp0, degp1)


def _tc_layer_body(a0_ref, a1_ref, hs_ref, dinv_ref, b_ref, w_ref, out_ref):
    dinv = dinv_ref[...]
    h = jnp.maximum(
        (a0_ref[...] + a1_ref[...] + hs_ref[...]) * dinv + b_ref[...], 0.0)
    out_ref[...] = (
        jnp.dot(h, w_ref[...], preferred_element_type=jnp.float32) * dinv)


def _tc_layer(acc0, acc1, hs, dinv, b, w):
    return pl.pallas_call(
        _tc_layer_body,
        grid=(NP // _BM,),
        in_specs=[
            pl.BlockSpec((_BM, D), lambda i: (i, 0)),
            pl.BlockSpec((_BM, D), lambda i: (i, 0)),
            pl.BlockSpec((_BM, D), lambda i: (i, 0)),
            pl.BlockSpec((_BM, 1), lambda i: (i, 0)),
            pl.BlockSpec((1, D), lambda i: (0, 0)),
            pl.BlockSpec((D, D), lambda i: (0, 0)),
        ],
        out_specs=pl.BlockSpec((_BM, D), lambda i: (i, 0)),
        out_shape=jax.ShapeDtypeStruct((NP, D), jnp.float32),
    )(acc0, acc1, hs, dinv, b, w)


def _tc_node_ab_body(a0_ref, a1_ref, hs_ref, dinv_ref, b_ref, wr_ref, wc_ref,
                     aout_ref, bout_ref):
    h = jnp.maximum(
        (a0_ref[...] + a1_ref[...] + hs_ref[...]) * dinv_ref[...] + b_ref[...],
        0.0)
    aout_ref[...] = jnp.dot(h, wr_ref[...], preferred_element_type=jnp.float32)
    bout_ref[...] = jnp.dot(h, wc_ref[...], preferred_element_type=jnp.float32)


def _tc_node_ab(acc0, acc1, hs, dinv, b, wr, wc):
    return pl.pallas_call(
        _tc_node_ab_body,
        grid=(NP // _BM,),
        in_specs=[
            pl.BlockSpec((_BM, D), lambda i: (i, 0)),
            pl.BlockSpec((_BM, D), lambda i: (i, 0)),
            pl.BlockSpec((_BM, D), lambda i: (i, 0)),
            pl.BlockSpec((_BM, 1), lambda i: (i, 0)),
            pl.BlockSpec((1, D), lambda i: (0, 0)),
            pl.BlockSpec((D, D), lambda i: (0, 0)),
            pl.BlockSpec((D, D), lambda i: (0, 0)),
        ],
        out_specs=[
            pl.BlockSpec((_BM, D), lambda i: (i, 0)),
            pl.BlockSpec((_BM, D), lambda i: (i, 0)),
        ],
        out_shape=[
            jax.ShapeDtypeStruct((NP, D), jnp.float32),
            jax.ShapeDtypeStruct((NP, D), jnp.float32),
        ],
    )(acc0, acc1, hs, dinv, b, wr, wc)


def _tc_edge_c_body(ea_ref, w_ref, b_ref, out_ref):
    out_ref[...] = (
        jnp.dot(ea_ref[...], w_ref[...], preferred_element_type=jnp.float32)
        + b_ref[...])


def _tc_edge_c(edge_attr, we, bm1):
    return pl.pallas_call(
        _tc_edge_c_body,
        grid=(E // _BE,),
        in_specs=[
            pl.BlockSpec((_BE, DE), lambda i: (i, 0)),
            pl.BlockSpec((DE, D), lambda i: (0, 0)),
            pl.BlockSpec((1, D), lambda i: (0, 0)),
        ],
        out_specs=pl.BlockSpec((_BE, D), lambda i: (i, 0)),
        out_shape=jax.ShapeDtypeStruct((E, D), jnp.float32),
    )(edge_attr, we, bm1)


def kernel(x, edge_index, edge_attr, W1, b1, W2, b2, Wm1, bm1, Wm2, bm2):
    src2 = edge_index[0].astype(jnp.int32).reshape(NW, EPW)
    src = src2.reshape(NW, NCHUNK, CK)
    dst = edge_index[1].astype(jnp.int32).reshape(NW, NCHUNK, CK)
    xp = jnp.pad(x, ((0, NP - N), (0, 0)))
    zeros128 = jnp.zeros((NP, D), jnp.float32)
    ones128 = jnp.ones((CK, D), jnp.float32)

    degp = _sc_degree(dst, ones128, zeros128)
    hs0, dinv = _tc_hs0(xp, W1, degp[0], degp[1])
    accp1 = _sc_aggregate(hs0, src2, dst, zeros128)
    hs1 = _tc_layer(accp1[0], accp1[1], hs0, dinv, b1.reshape(1, D), W2)
    accp2 = _sc_aggregate(hs1, src2, dst, zeros128)
    a_t, b_t = _tc_node_ab(accp2[0], accp2[1], hs1, dinv, b2.reshape(1, D),
                           Wm1[:D], Wm1[D:2 * D])
    c_t = _tc_edge_c(edge_attr, Wm1[2 * D:], bm1.reshape(1, D))
    logits = _sc_edge_mlp(a_t, b_t, c_t, src, dst, Wm2[:, 0],
                          jnp.full((16,), bm2[0], jnp.float32))
    return logits


# speedup vs baseline: 1.1609x; 1.0774x over previous
import functools

import jax
import jax.numpy as jnp
from jax import lax
from jax.experimental import pallas as pl
from jax.experimental.pallas import tpu as pltpu
from jax.experimental.pallas import tpu_sc as plsc

N = 10000
NP = 10240
E = 320000
D = 128
DE = 16

NC = 2
NS = 16
NW = NC * NS
EPW = E // NW
CK = 80
NCHUNK = EPW // CK
RPS = NP // NS

_mesh = plsc.VectorSubcoreMesh(
    core_axis_name="c", subcore_axis_name="s", num_cores=NC, num_subcores=NS)


@functools.partial(
    pl.kernel,
    out_type=jax.ShapeDtypeStruct((NC, NP, D), jnp.float32),
    mesh=_mesh,
    scratch_types=[
        pltpu.VMEM((NCHUNK, CK), jnp.int32),
        pltpu.VMEM((CK, D), jnp.float32),
        pltpu.VMEM_SHARED((NP, D), jnp.float32),
        pltpu.SemaphoreType.DMA,
    ],
)
def _sc_degree(dst_hbm, ones_hbm, zeros_hbm, out_hbm, idx_v, ones_v, acc_sh,
               sem):
    c = lax.axis_index("c")
    s = lax.axis_index("s")
    wid = c * NS + s
    pltpu.sync_copy(zeros_hbm.at[pl.ds(s * RPS, RPS)],
                    acc_sh.at[pl.ds(s * RPS, RPS)])
    pltpu.sync_copy(dst_hbm.at[wid], idx_v)
    pltpu.sync_copy(ones_hbm, ones_v)
    plsc.subcore_barrier()

    def fire(m, carry):
        pltpu.async_copy(ones_v, acc_sh.at[idx_v.at[m]], sem, add=True)
        return carry

    def drain(m, carry):
        pltpu.make_async_copy(ones_v, acc_sh.at[idx_v.at[m]], sem).wait()
        return carry

    lax.fori_loop(0, NCHUNK, fire, 0)
    lax.fori_loop(0, NCHUNK, drain, 0)
    plsc.subcore_barrier()
    pltpu.sync_copy(acc_sh.at[pl.ds(s * RPS, RPS)],
                    out_hbm.at[c, pl.ds(s * RPS, RPS)])


@functools.partial(
    pl.kernel,
    out_type=jax.ShapeDtypeStruct((NC, NP, D), jnp.float32),
    mesh=_mesh,
    scratch_types=[
        pltpu.VMEM((EPW,), jnp.int32),
        pltpu.VMEM((NCHUNK, CK), jnp.int32),
        pltpu.VMEM((2, CK, D), jnp.float32),
        pltpu.VMEM_SHARED((NP, D), jnp.float32),
        pltpu.SemaphoreType.DMA((2,)),
    ],
)
def _sc_aggregate(hs_hbm, src_hbm, dst_hbm, zeros_hbm, out_hbm,
                  sidx_v, didx_v, rows_v, acc_sh, sg):
    c = lax.axis_index("c")
    s = lax.axis_index("s")
    wid = c * NS + s
    pltpu.sync_copy(zeros_hbm.at[pl.ds(s * RPS, RPS)],
                    acc_sh.at[pl.ds(s * RPS, RPS)])
    pltpu.sync_copy(src_hbm.at[wid], sidx_v)
    pltpu.sync_copy(dst_hbm.at[wid], didx_v)
    plsc.subcore_barrier()

    def issue_g(m, k):
        pltpu.async_copy(hs_hbm.at[sidx_v.at[pl.ds(m * CK, CK)]],
                         rows_v.at[k], sg.at[k])

    def wait_g(m, k):
        pltpu.make_async_copy(hs_hbm.at[sidx_v.at[pl.ds(m * CK, CK)]],
                              rows_v.at[k], sg.at[k]).wait()

    for k in range(2):
        issue_g(k, k)

    def body(mm, carry):
        for k in range(2):
            cc = 2 * mm + k
            wait_g(cc, k)
            pltpu.sync_copy(rows_v.at[k], acc_sh.at[didx_v.at[cc]], add=True)

            @pl.when(cc + 2 < NCHUNK)
            def _(cc=cc, k=k):
                issue_g(cc + 2, k)
        return carry

    lax.fori_loop(0, NCHUNK // 2, body, 0)
    for cc in range(2 * (NCHUNK // 2), NCHUNK):
        k = cc % 2
        wait_g(cc, k)
        pltpu.sync_copy(rows_v.at[k], acc_sh.at[didx_v.at[cc]], add=True)
    plsc.subcore_barrier()
    pltpu.sync_copy(acc_sh.at[pl.ds(s * RPS, RPS)],
                    out_hbm.at[c, pl.ds(s * RPS, RPS)])


@functools.partial(
    pl.kernel,
    out_type=jax.ShapeDtypeStruct((E,), jnp.float32),
    mesh=_mesh,
    compiler_params=pltpu.CompilerParams(needs_layout_passes=False),
    scratch_types=[
        pltpu.VMEM((NCHUNK, CK), jnp.int32),
        pltpu.VMEM((NCHUNK, CK), jnp.int32),
        pltpu.VMEM((2, CK, D), jnp.float32),
        pltpu.VMEM((EPW,), jnp.float32),
        pltpu.VMEM((D,), jnp.float32),
        pltpu.VMEM((16,), jnp.float32),
        pltpu.SemaphoreType.DMA((2,)),
        pltpu.SemaphoreType.DMA((2,)),
    ],
)
def _sc_edge_mlp(a_hbm, b_hbm, c_hbm, row_hbm, col_hbm, w2_hbm,
                 bm2_hbm, out_hbm, ridx_v, cidx_v, av, ov, w2v, bm2v,
                 sa, sadd):
    c = lax.axis_index("c")
    s = lax.axis_index("s")
    wid = c * NS + s
    ebase = pl.multiple_of(wid * EPW, 8)
    pltpu.sync_copy(row_hbm.at[wid], ridx_v)
    pltpu.sync_copy(col_hbm.at[wid], cidx_v)
    pltpu.sync_copy(w2_hbm, w2v)
    pltpu.sync_copy(bm2_hbm, bm2v)
    w = [w2v[pl.ds(16 * d, 16)] for d in range(D // 16)]
    bmv = bm2v[...]
    lanes = lax.iota(jnp.int32, 16)

    def issue_a(m, k):
        off = pl.multiple_of(ebase + m * CK, 8)
        pltpu.async_copy(c_hbm.at[pl.ds(off, CK)], av.at[k], sa.at[k])

    def wait_a(m, k):
        off = pl.multiple_of(ebase + m * CK, 8)
        pltpu.make_async_copy(c_hbm.at[pl.ds(off, CK)], av.at[k],
                              sa.at[k]).wait()

    def issue_adds(m, k):
        pltpu.async_copy(a_hbm.at[ridx_v.at[m]], av.at[k], sadd.at[k],
                         add=True)
        pltpu.async_copy(b_hbm.at[cidx_v.at[m]], av.at[k], sadd.at[k],
                         add=True)

    def wait_adds(m, k):
        for _ in range(2):
            pltpu.make_async_copy(b_hbm.at[cidx_v.at[m]], av.at[k],
                                  sadd.at[k]).wait()

    def compute(m, k):
        def group(g, icarry):
            parts = []
            for q in range(4):
                r = jnp.zeros((16,), jnp.float32)
                for l in range(4 * q, 4 * q + 4):
                    j = g * 16 + l
                    acc = jnp.zeros((16,), jnp.float32)
                    for d in range(D // 16):
                        sl = pl.ds(16 * d, 16)
                        t = av[k, j, sl]
                        acc = acc + jnp.maximum(t, 0.0) * w[d]
                    s = jnp.sum(acc)
                    r = jnp.where(lanes == l, r + s, r)
                parts.append(r)
            res = bmv + (parts[0] + parts[1]) + (parts[2] + parts[3])
            ov[pl.ds(m * CK + g * 16, 16)] = res
            return icarry

        lax.fori_loop(0, CK // 16, group, 0)

    issue_a(0, 0)
    wait_a(0, 0)
    issue_adds(0, 0)
    issue_a(1, 1)

    def body(mm, carry):
        for k in range(2):
            m = 2 * mm + k
            nxt = m + 1

            @pl.when(nxt < NCHUNK)
            def _(nxt=nxt, k=k):
                wait_a(nxt, 1 - k)
                issue_adds(nxt, 1 - k)

            @pl.when(m < NCHUNK)
            def _(m=m, k=k):
                wait_adds(m, k)
                compute(m, k)

            @pl.when(nxt + 1 < NCHUNK)
            def _(nxt=nxt, k=k):
                issue_a(nxt + 1, k)
        return carry

    lax.fori_loop(0, (NCHUNK + 1) // 2, body, 0)
    pltpu.sync_copy(ov, out_hbm.at[pl.ds(ebase, EPW)])


_BM = 640
_BE = 2000


def _tc_hs0_body(x_ref, w_ref, d0_ref, d1_ref, hs_ref, dinv_ref):
    deg = d0_ref[:, 0:1] + d1_ref[:, 0:1] + 1.0
    dinv = lax.rsqrt(deg)
    h = jnp.dot(x_ref[...], w_ref[...], preferred_element_type=jnp.float32)
    hs_ref[...] = h * dinv
    dinv_ref[...] = dinv


def _tc_hs0(x, w1, degp0, degp1):
    return pl.pallas_call(
        _tc_hs0_body,
        grid=(NP // _BM,),
        in_specs=[
            pl.BlockSpec((_BM, D), lambda i: (i, 0)),
            pl.BlockSpec((D, D), lambda i: (0, 0)),
            pl.BlockSpec((_BM, D), lambda i: (i, 0)),
            pl.BlockSpec((_BM, D), lambda i: (i, 0)),
        ],
        out_specs=[
            pl.BlockSpec((_BM, D), lambda i: (i, 0)),
            pl.BlockSpec((_BM, 1), lambda i: (i, 0)),
        ],
        out_shape=[
            jax.ShapeDtypeStruct((NP, D), jnp.float32),
            jax.ShapeDtypeStruct((NP, 1), jnp.float32),
        ],
    )(x, w1, degp0, degp1)


def _tc_layer_body(a0_ref, a1_ref, hs_ref, dinv_ref, b_ref, w_ref, out_ref):
    dinv = dinv_ref[...]
    h = jnp.maximum(
        (a0_ref[...] + a1_ref[...] + hs_ref[...]) * dinv + b_ref[...], 0.0)
    out_ref[...] = (
        jnp.dot(h, w_ref[...], preferred_element_type=jnp.float32) * dinv)


def _tc_layer(acc0, acc1, hs, dinv, b, w):
    return pl.pallas_call(
        _tc_layer_body,
        grid=(NP // _BM,),
        in_specs=[
            pl.BlockSpec((_BM, D), lambda i: (i, 0)),
            pl.BlockSpec((_BM, D), lambda i: (i, 0)),
            pl.BlockSpec((_BM, D), lambda i: (i, 0)),
            pl.BlockSpec((_BM, 1), lambda i: (i, 0)),
            pl.BlockSpec((1, D), lambda i: (0, 0)),
            pl.BlockSpec((D, D), lambda i: (0, 0)),
        ],
        out_specs=pl.BlockSpec((_BM, D), lambda i: (i, 0)),
        out_shape=jax.ShapeDtypeStruct((NP, D), jnp.float32),
    )(acc0, acc1, hs, dinv, b, w)


def _tc_node_ab_body(a0_ref, a1_ref, hs_ref, dinv_ref, b_ref, wr_ref, wc_ref,
                     aout_ref, bout_ref):
    h = jnp.maximum(
        (a0_ref[...] + a1_ref[...] + hs_ref[...]) * dinv_ref[...] + b_ref[...],
        0.0)
    aout_ref[...] = jnp.dot(h, wr_ref[...], preferred_element_type=jnp.float32)
    bout_ref[...] = jnp.dot(h, wc_ref[...], preferred_element_type=jnp.float32)


def _tc_node_ab(acc0, acc1, hs, dinv, b, wr, wc):
    return pl.pallas_call(
        _tc_node_ab_body,
        grid=(NP // _BM,),
        in_specs=[
            pl.BlockSpec((_BM, D), lambda i: (i, 0)),
            pl.BlockSpec((_BM, D), lambda i: (i, 0)),
            pl.BlockSpec((_BM, D), lambda i: (i, 0)),
            pl.BlockSpec((_BM, 1), lambda i: (i, 0)),
            pl.BlockSpec((1, D), lambda i: (0, 0)),
            pl.BlockSpec((D, D), lambda i: (0, 0)),
            pl.BlockSpec((D, D), lambda i: (0, 0)),
        ],
        out_specs=[
            pl.BlockSpec((_BM, D), lambda i: (i, 0)),
            pl.BlockSpec((_BM, D), lambda i: (i, 0)),
        ],
        out_shape=[
            jax.ShapeDtypeStruct((NP, D), jnp.float32),
            jax.ShapeDtypeStruct((NP, D), jnp.float32),
        ],
    )(acc0, acc1, hs, dinv, b, wr, wc)


def _tc_edge_c_body(ea_ref, w_ref, b_ref, out_ref):
    out_ref[...] = (
        jnp.dot(ea_ref[...], w_ref[...], preferred_element_type=jnp.float32)
        + b_ref[...])


def _tc_edge_c(edge_attr, we, bm1):
    return pl.pallas_call(
        _tc_edge_c_body,
        grid=(E // _BE,),
        in_specs=[
            pl.BlockSpec((_BE, DE), lambda i: (i, 0)),
            pl.BlockSpec((DE, D), lambda i: (0, 0)),
            pl.BlockSpec((1, D), lambda i: (0, 0)),
        ],
        out_specs=pl.BlockSpec((_BE, D), lambda i: (i, 0)),
        out_shape=jax.ShapeDtypeStruct((E, D), jnp.float32),
    )(edge_attr, we, bm1)


def kernel(x, edge_index, edge_attr, W1, b1, W2, b2, Wm1, bm1, Wm2, bm2):
    src2 = edge_index[0].astype(jnp.int32).reshape(NW, EPW)
    src = src2.reshape(NW, NCHUNK, CK)
    dst = edge_index[1].astype(jnp.int32).reshape(NW, NCHUNK, CK)
    xp = jnp.pad(x, ((0, NP - N), (0, 0)))
    zeros128 = jnp.zeros((NP, D), jnp.float32)
    ones128 = jnp.ones((CK, D), jnp.float32)

    degp = _sc_degree(dst, ones128, zeros128)
    hs0, dinv = _tc_hs0(xp, W1, degp[0], degp[1])
    accp1 = _sc_aggregate(hs0, src2, dst, zeros128)
    hs1 = _tc_layer(accp1[0], accp1[1], hs0, dinv, b1.reshape(1, D), W2)
    accp2 = _sc_aggregate(hs1, src2, dst, zeros128)
    a_t, b_t = _tc_node_ab(accp2[0], accp2[1], hs1, dinv, b2.reshape(1, D),
                           Wm1[:D], Wm1[D:2 * D])
    c_t = _tc_edge_c(edge_attr, Wm1[2 * D:], bm1.reshape(1, D))
    logits = _sc_edge_mlp(a_t, b_t, c_t, src, dst, Wm2[:, 0],
                          jnp.full((16,), bm2[0], jnp.float32))
    return logits


# speedup vs baseline: 1.1651x; 1.0037x over previous
import functools

import jax
import jax.numpy as jnp
from jax import lax
from jax.experimental import pallas as pl
from jax.experimental.pallas import tpu as pltpu
from jax.experimental.pallas import tpu_sc as plsc

N = 10000
NP = 10240
E = 320000
D = 128
DE = 16

NC = 2
NS = 16
NW = NC * NS
EPW = E // NW
CK = 80
NCHUNK = EPW // CK
RPS = NP // NS

_mesh = plsc.VectorSubcoreMesh(
    core_axis_name="c", subcore_axis_name="s", num_cores=NC, num_subcores=NS)


@functools.partial(
    pl.kernel,
    out_type=jax.ShapeDtypeStruct((NC, NP, D), jnp.float32),
    mesh=_mesh,
    scratch_types=[
        pltpu.VMEM((NCHUNK, CK), jnp.int32),
        pltpu.VMEM((CK, D), jnp.float32),
        pltpu.VMEM_SHARED((NP, D), jnp.float32),
        pltpu.SemaphoreType.DMA,
    ],
)
def _sc_degree(dst_hbm, ones_hbm, zeros_hbm, out_hbm, idx_v, ones_v, acc_sh,
               sem):
    c = lax.axis_index("c")
    s = lax.axis_index("s")
    wid = c * NS + s
    pltpu.sync_copy(zeros_hbm.at[pl.ds(s * RPS, RPS)],
                    acc_sh.at[pl.ds(s * RPS, RPS)])
    pltpu.sync_copy(dst_hbm.at[wid], idx_v)
    pltpu.sync_copy(ones_hbm, ones_v)
    plsc.subcore_barrier()

    def fire(m, carry):
        pltpu.async_copy(ones_v, acc_sh.at[idx_v.at[m]], sem, add=True)
        return carry

    def drain(m, carry):
        pltpu.make_async_copy(ones_v, acc_sh.at[idx_v.at[m]], sem).wait()
        return carry

    lax.fori_loop(0, NCHUNK, fire, 0)
    lax.fori_loop(0, NCHUNK, drain, 0)
    plsc.subcore_barrier()
    pltpu.sync_copy(acc_sh.at[pl.ds(s * RPS, RPS)],
                    out_hbm.at[c, pl.ds(s * RPS, RPS)])


@functools.partial(
    pl.kernel,
    out_type=jax.ShapeDtypeStruct((NC, NP, D), jnp.float32),
    mesh=_mesh,
    scratch_types=[
        pltpu.VMEM((EPW,), jnp.int32),
        pltpu.VMEM((NCHUNK, CK), jnp.int32),
        pltpu.VMEM((2, CK, D), jnp.float32),
        pltpu.VMEM_SHARED((NP, D), jnp.float32),
        pltpu.SemaphoreType.DMA((2,)),
    ],
)
def _sc_aggregate(hs_hbm, src_hbm, dst_hbm, zeros_hbm, out_hbm,
                  sidx_v, didx_v, rows_v, acc_sh, sg):
    c = lax.axis_index("c")
    s = lax.axis_index("s")
    wid = c * NS + s
    pltpu.sync_copy(zeros_hbm.at[pl.ds(s * RPS, RPS)],
                    acc_sh.at[pl.ds(s * RPS, RPS)])
    pltpu.sync_copy(src_hbm.at[wid], sidx_v)
    pltpu.sync_copy(dst_hbm.at[wid], didx_v)
    plsc.subcore_barrier()

    def issue_g(m, k):
        pltpu.async_copy(hs_hbm.at[sidx_v.at[pl.ds(m * CK, CK)]],
                         rows_v.at[k], sg.at[k])

    def wait_g(m, k):
        pltpu.make_async_copy(hs_hbm.at[sidx_v.at[pl.ds(m * CK, CK)]],
                              rows_v.at[k], sg.at[k]).wait()

    for k in range(2):
        issue_g(k, k)

    def body(mm, carry):
        for k in range(2):
            cc = 2 * mm + k
            wait_g(cc, k)
            pltpu.sync_copy(rows_v.at[k], acc_sh.at[didx_v.at[cc]], add=True)

            @pl.when(cc + 2 < NCHUNK)
            def _(cc=cc, k=k):
                issue_g(cc + 2, k)
        return carry

    lax.fori_loop(0, NCHUNK // 2, body, 0)
    for cc in range(2 * (NCHUNK // 2), NCHUNK):
        k = cc % 2
        wait_g(cc, k)
        pltpu.sync_copy(rows_v.at[k], acc_sh.at[didx_v.at[cc]], add=True)
    plsc.subcore_barrier()
    pltpu.sync_copy(acc_sh.at[pl.ds(s * RPS, RPS)],
                    out_hbm.at[c, pl.ds(s * RPS, RPS)])


@functools.partial(
    pl.kernel,
    out_type=jax.ShapeDtypeStruct((E,), jnp.float32),
    mesh=_mesh,
    compiler_params=pltpu.CompilerParams(needs_layout_passes=False),
    scratch_types=[
        pltpu.VMEM((NCHUNK, CK), jnp.int32),
        pltpu.VMEM((NCHUNK, CK), jnp.int32),
        pltpu.VMEM((2, CK, D), jnp.float32),
        pltpu.VMEM((EPW,), jnp.float32),
        pltpu.VMEM((D,), jnp.float32),
        pltpu.VMEM((16,), jnp.float32),
        pltpu.SemaphoreType.DMA((2,)),
        pltpu.SemaphoreType.DMA((2,)),
    ],
)
def _sc_edge_mlp(a_hbm, b_hbm, c_hbm, row_hbm, col_hbm, w2_hbm,
                 bm2_hbm, out_hbm, ridx_v, cidx_v, av, ov, w2v, bm2v,
                 sa, sadd):
    c = lax.axis_index("c")
    s = lax.axis_index("s")
    wid = c * NS + s
    ebase = pl.multiple_of(wid * EPW, 8)
    pltpu.sync_copy(row_hbm.at[wid], ridx_v)
    pltpu.sync_copy(col_hbm.at[wid], cidx_v)
    pltpu.sync_copy(w2_hbm, w2v)
    pltpu.sync_copy(bm2_hbm, bm2v)
    w = [w2v[pl.ds(16 * d, 16)] for d in range(D // 16)]
    bmv = bm2v[...]
    lanes = lax.iota(jnp.int32, 16)

    def issue_a(m, k):
        off = pl.multiple_of(ebase + m * CK, 8)
        pltpu.async_copy(c_hbm.at[pl.ds(off, CK)], av.at[k], sa.at[k])

    def wait_a(m, k):
        off = pl.multiple_of(ebase + m * CK, 8)
        pltpu.make_async_copy(c_hbm.at[pl.ds(off, CK)], av.at[k],
                              sa.at[k]).wait()

    def issue_adds(m, k):
        pltpu.async_copy(a_hbm.at[ridx_v.at[m]], av.at[k], sadd.at[k],
                         add=True)
        pltpu.async_copy(b_hbm.at[cidx_v.at[m]], av.at[k], sadd.at[k],
                         add=True)

    def wait_adds(m, k):
        for _ in range(2):
            pltpu.make_async_copy(b_hbm.at[cidx_v.at[m]], av.at[k],
                                  sadd.at[k]).wait()

    def compute(m, k):
        def group(g, icarry):
            res = bmv
            for l in range(16):
                j = g * 16 + l
                acc = jnp.zeros((16,), jnp.float32)
                for d in range(D // 16):
                    sl = pl.ds(16 * d, 16)
                    t = av[k, j, sl]
                    acc = acc + jnp.maximum(t, 0.0) * w[d]
                s = jnp.sum(acc)
                res = jnp.where(lanes == l, res + s, res)
            ov[pl.ds(m * CK + g * 16, 16)] = res
            return icarry

        lax.fori_loop(0, CK // 16, group, 0)

    issue_a(0, 0)
    wait_a(0, 0)
    issue_adds(0, 0)
    issue_a(1, 1)

    def body(mm, carry):
        for k in range(2):
            m = 2 * mm + k
            nxt = m + 1

            @pl.when(nxt < NCHUNK)
            def _(nxt=nxt, k=k):
                wait_a(nxt, 1 - k)
                issue_adds(nxt, 1 - k)

            @pl.when(m < NCHUNK)
            def _(m=m, k=k):
                wait_adds(m, k)
                compute(m, k)

            @pl.when(nxt + 1 < NCHUNK)
            def _(nxt=nxt, k=k):
                issue_a(nxt + 1, k)
        return carry

    lax.fori_loop(0, (NCHUNK + 1) // 2, body, 0)
    pltpu.sync_copy(ov, out_hbm.at[pl.ds(ebase, EPW)])


_BM = 640
_BE = 2000


def _tc_hs0_body(x_ref, w_ref, d0_ref, d1_ref, hs_ref, dinv_ref):
    deg = d0_ref[:, 0:1] + d1_ref[:, 0:1] + 1.0
    dinv = lax.rsqrt(deg)
    h = jnp.dot(x_ref[...], w_ref[...], preferred_element_type=jnp.float32)
    hs_ref[...] = h * dinv
    dinv_ref[...] = dinv


def _tc_hs0(x, w1, degp0, degp1):
    return pl.pallas_call(
        _tc_hs0_body,
        grid=(NP // _BM,),
        in_specs=[
            pl.BlockSpec((_BM, D), lambda i: (i, 0)),
            pl.BlockSpec((D, D), lambda i: (0, 0)),
            pl.BlockSpec((_BM, D), lambda i: (i, 0)),
            pl.BlockSpec((_BM, D), lambda i: (i, 0)),
        ],
        out_specs=[
            pl.BlockSpec((_BM, D), lambda i: (i, 0)),
            pl.BlockSpec((_BM, 1), lambda i: (i, 0)),
        ],
        out_shape=[
            jax.ShapeDtypeStruct((NP, D), jnp.float32),
            jax.ShapeDtypeStruct((NP, 1), jnp.float32),
        ],
    )(x, w1, degp0, degp1)


def _tc_layer_body(a0_ref, a1_ref, hs_ref, dinv_ref, b_ref, w_ref, out_ref):
    dinv = dinv_ref[...]
    h = jnp.maximum(
        (a0_ref[...] + a1_ref[...] + hs_ref[...]) * dinv + b_ref[...], 0.0)
    out_ref[...] = (
        jnp.dot(h, w_ref[...], preferred_element_type=jnp.float32) * dinv)


def _tc_layer(acc0, acc1, hs, dinv, b, w):
    return pl.pallas_call(
        _tc_layer_body,
        grid=(NP // _BM,),
        in_specs=[
            pl.BlockSpec((_BM, D), lambda i: (i, 0)),
            pl.BlockSpec((_BM, D), lambda i: (i, 0)),
            pl.BlockSpec((_BM, D), lambda i: (i, 0)),
            pl.BlockSpec((_BM, 1), lambda i: (i, 0)),
            pl.BlockSpec((1, D), lambda i: (0, 0)),
            pl.BlockSpec((D, D), lambda i: (0, 0)),
        ],
        out_specs=pl.BlockSpec((_BM, D), lambda i: (i, 0)),
        out_shape=jax.ShapeDtypeStruct((NP, D), jnp.float32),
    )(acc0, acc1, hs, dinv, b, w)


def _tc_node_ab_body(a0_ref, a1_ref, hs_ref, dinv_ref, b_ref, wr_ref, wc_ref,
                     aout_ref, bout_ref):
    h = jnp.maximum(
        (a0_ref[...] + a1_ref[...] + hs_ref[...]) * dinv_ref[...] + b_ref[...],
        0.0)
    aout_ref[...] = jnp.dot(h, wr_ref[...], preferred_element_type=jnp.float32)
    bout_ref[...] = jnp.dot(h, wc_ref[...], preferred_element_type=jnp.float32)


def _tc_node_ab(acc0, acc1, hs, dinv, b, wr, wc):
    return pl.pallas_call(
        _tc_node_ab_body,
        grid=(NP // _BM,),
        in_specs=[
            pl.BlockSpec((_BM, D), lambda i: (i, 0)),
            pl.BlockSpec((_BM, D), lambda i: (i, 0)),
            pl.BlockSpec((_BM, D), lambda i: (i, 0)),
            pl.BlockSpec((_BM, 1), lambda i: (i, 0)),
            pl.BlockSpec((1, D), lambda i: (0, 0)),
            pl.BlockSpec((D, D), lambda i: (0, 0)),
            pl.BlockSpec((D, D), lambda i: (0, 0)),
        ],
        out_specs=[
            pl.BlockSpec((_BM, D), lambda i: (i, 0)),
            pl.BlockSpec((_BM, D), lambda i: (i, 0)),
        ],
        out_shape=[
            jax.ShapeDtypeStruct((NP, D), jnp.float32),
            jax.ShapeDtypeStruct((NP, D), jnp.float32),
        ],
    )(acc0, acc1, hs, dinv, b, wr, wc)


def _tc_edge_c_body(ea_ref, w_ref, b_ref, out_ref):
    out_ref[...] = (
        jnp.dot(ea_ref[...], w_ref[...], preferred_element_type=jnp.float32)
        + b_ref[...])


def _tc_edge_c(edge_attr, we, bm1):
    return pl.pallas_call(
        _tc_edge_c_body,
        grid=(E // _BE,),
        in_specs=[
            pl.BlockSpec((_BE, DE), lambda i: (i, 0)),
            pl.BlockSpec((DE, D), lambda i: (0, 0)),
            pl.BlockSpec((1, D), lambda i: (0, 0)),
        ],
        out_specs=pl.BlockSpec((_BE, D), lambda i: (i, 0)),
        out_shape=jax.ShapeDtypeStruct((E, D), jnp.float32),
    )(edge_attr, we, bm1)


def kernel(x, edge_index, edge_attr, W1, b1, W2, b2, Wm1, bm1, Wm2, bm2):
    src2 = edge_index[0].astype(jnp.int32).reshape(NW, EPW)
    src = src2.reshape(NW, NCHUNK, CK)
    dst = edge_index[1].astype(jnp.int32).reshape(NW, NCHUNK, CK)
    xp = jnp.pad(x, ((0, NP - N), (0, 0)))
    zeros128 = jnp.zeros((NP, D), jnp.float32)
    ones128 = jnp.ones((CK, D), jnp.float32)

    degp = _sc_degree(dst, ones128, zeros128)
    hs0, dinv = _tc_hs0(xp, W1, degp[0], degp[1])
    accp1 = _sc_aggregate(hs0, src2, dst, zeros128)
    hs1 = _tc_layer(accp1[0], accp1[1], hs0, dinv, b1.reshape(1, D), W2)
    accp2 = _sc_aggregate(hs1, src2, dst, zeros128)
    a_t, b_t = _tc_node_ab(accp2[0], accp2[1], hs1, dinv, b2.reshape(1, D),
                           Wm1[:D], Wm1[D:2 * D])
    c_t = _tc_edge_c(edge_attr, Wm1[2 * D:], bm1.reshape(1, D))
    logits = _sc_edge_mlp(a_t, b_t, c_t, src, dst, Wm2[:, 0],
                          jnp.full((16,), bm2[0], jnp.float32))
    return logits


# speedup vs baseline: 1.2178x; 1.0452x over previous
import functools

import jax
import jax.numpy as jnp
from jax import lax
from jax.experimental import pallas as pl
from jax.experimental.pallas import tpu as pltpu
from jax.experimental.pallas import tpu_sc as plsc

N = 10000
NP = 10240
E = 320000
D = 128
DE = 16

NC = 2
NS = 16
NW = NC * NS
EPW = E // NW
CK = 80
NCHUNK = EPW // CK
RPS = NP // NS

_mesh = plsc.VectorSubcoreMesh(
    core_axis_name="c", subcore_axis_name="s", num_cores=NC, num_subcores=NS)


@functools.partial(
    pl.kernel,
    out_type=jax.ShapeDtypeStruct((NC, NP, D), jnp.float32),
    mesh=_mesh,
    scratch_types=[
        pltpu.VMEM((NCHUNK, CK), jnp.int32),
        pltpu.VMEM((CK, D), jnp.float32),
        pltpu.VMEM_SHARED((NP, D), jnp.float32),
        pltpu.SemaphoreType.DMA,
    ],
)
def _sc_degree(dst_hbm, ones_hbm, zeros_hbm, out_hbm, idx_v, ones_v, acc_sh,
               sem):
    c = lax.axis_index("c")
    s = lax.axis_index("s")
    wid = c * NS + s
    pltpu.sync_copy(zeros_hbm.at[pl.ds(s * RPS, RPS)],
                    acc_sh.at[pl.ds(s * RPS, RPS)])
    pltpu.sync_copy(dst_hbm.at[wid], idx_v)
    pltpu.sync_copy(ones_hbm, ones_v)
    plsc.subcore_barrier()

    def fire(m, carry):
        pltpu.async_copy(ones_v, acc_sh.at[idx_v.at[m]], sem, add=True)
        return carry

    def drain(m, carry):
        pltpu.make_async_copy(ones_v, acc_sh.at[idx_v.at[m]], sem).wait()
        return carry

    lax.fori_loop(0, NCHUNK, fire, 0)
    lax.fori_loop(0, NCHUNK, drain, 0)
    plsc.subcore_barrier()
    pltpu.sync_copy(acc_sh.at[pl.ds(s * RPS, RPS)],
                    out_hbm.at[c, pl.ds(s * RPS, RPS)])


@functools.partial(
    pl.kernel,
    out_type=jax.ShapeDtypeStruct((NC, NP, D), jnp.float32),
    mesh=_mesh,
    scratch_types=[
        pltpu.VMEM((EPW,), jnp.int32),
        pltpu.VMEM((NCHUNK, CK), jnp.int32),
        pltpu.VMEM((2, CK, D), jnp.float32),
        pltpu.VMEM_SHARED((NP, D), jnp.float32),
        pltpu.SemaphoreType.DMA((2,)),
    ],
)
def _sc_aggregate(hs_hbm, src_hbm, dst_hbm, zeros_hbm, out_hbm,
                  sidx_v, didx_v, rows_v, acc_sh, sg):
    c = lax.axis_index("c")
    s = lax.axis_index("s")
    wid = c * NS + s
    pltpu.sync_copy(zeros_hbm.at[pl.ds(s * RPS, RPS)],
                    acc_sh.at[pl.ds(s * RPS, RPS)])
    pltpu.sync_copy(src_hbm.at[wid], sidx_v)
    pltpu.sync_copy(dst_hbm.at[wid], didx_v)
    plsc.subcore_barrier()

    def issue_g(m, k):
        pltpu.async_copy(hs_hbm.at[sidx_v.at[pl.ds(m * CK, CK)]],
                         rows_v.at[k], sg.at[k])

    def wait_g(m, k):
        pltpu.make_async_copy(hs_hbm.at[sidx_v.at[pl.ds(m * CK, CK)]],
                              rows_v.at[k], sg.at[k]).wait()

    for k in range(2):
        issue_g(k, k)

    def body(mm, carry):
        for k in range(2):
            cc = 2 * mm + k
            wait_g(cc, k)
            pltpu.sync_copy(rows_v.at[k], acc_sh.at[didx_v.at[cc]], add=True)

            @pl.when(cc + 2 < NCHUNK)
            def _(cc=cc, k=k):
                issue_g(cc + 2, k)
        return carry

    lax.fori_loop(0, NCHUNK // 2, body, 0)
    for cc in range(2 * (NCHUNK // 2), NCHUNK):
        k = cc % 2
        wait_g(cc, k)
        pltpu.sync_copy(rows_v.at[k], acc_sh.at[didx_v.at[cc]], add=True)
    plsc.subcore_barrier()
    pltpu.sync_copy(acc_sh.at[pl.ds(s * RPS, RPS)],
                    out_hbm.at[c, pl.ds(s * RPS, RPS)])


@functools.partial(
    pl.kernel,
    out_type=jax.ShapeDtypeStruct((E,), jnp.float32),
    mesh=_mesh,
    compiler_params=pltpu.CompilerParams(needs_layout_passes=False),
    scratch_types=[
        pltpu.VMEM((NCHUNK, CK), jnp.int32),
        pltpu.VMEM((NCHUNK, CK), jnp.int32),
        pltpu.VMEM((4, CK, D), jnp.float32),
        pltpu.VMEM((EPW,), jnp.float32),
        pltpu.VMEM((D,), jnp.float32),
        pltpu.VMEM((16,), jnp.float32),
        pltpu.SemaphoreType.DMA((4,)),
        pltpu.SemaphoreType.DMA((4,)),
    ],
)
def _sc_edge_mlp(a_hbm, b_hbm, c_hbm, row_hbm, col_hbm, w2_hbm,
                 bm2_hbm, out_hbm, ridx_v, cidx_v, av, ov, w2v, bm2v,
                 sa, sadd):
    c = lax.axis_index("c")
    s = lax.axis_index("s")
    wid = c * NS + s
    ebase = pl.multiple_of(wid * EPW, 8)
    pltpu.sync_copy(row_hbm.at[wid], ridx_v)
    pltpu.sync_copy(col_hbm.at[wid], cidx_v)
    pltpu.sync_copy(w2_hbm, w2v)
    pltpu.sync_copy(bm2_hbm, bm2v)
    w = [w2v[pl.ds(16 * d, 16)] for d in range(D // 16)]
    bmv = bm2v[...]
    lanes = lax.iota(jnp.int32, 16)

    def issue_a(m, k):
        off = pl.multiple_of(ebase + m * CK, 8)
        pltpu.async_copy(c_hbm.at[pl.ds(off, CK)], av.at[k], sa.at[k])

    def wait_a(m, k):
        off = pl.multiple_of(ebase + m * CK, 8)
        pltpu.make_async_copy(c_hbm.at[pl.ds(off, CK)], av.at[k],
                              sa.at[k]).wait()

    def issue_adds(m, k):
        pltpu.async_copy(a_hbm.at[ridx_v.at[m]], av.at[k], sadd.at[k],
                         add=True)
        pltpu.async_copy(b_hbm.at[cidx_v.at[m]], av.at[k], sadd.at[k],
                         add=True)

    def wait_adds(m, k):
        for _ in range(2):
            pltpu.make_async_copy(b_hbm.at[cidx_v.at[m]], av.at[k],
                                  sadd.at[k]).wait()

    def compute(m, k):
        def group(g, icarry):
            res = bmv
            for l in range(16):
                j = g * 16 + l
                acc = jnp.zeros((16,), jnp.float32)
                for d in range(D // 16):
                    sl = pl.ds(16 * d, 16)
                    t = av[k, j, sl]
                    acc = acc + jnp.maximum(t, 0.0) * w[d]
                s = jnp.sum(acc)
                res = jnp.where(lanes == l, res + s, res)
            ov[pl.ds(m * CK + g * 16, 16)] = res
            return icarry

        lax.fori_loop(0, CK // 16, group, 0)

    issue_a(0, 0)
    issue_a(1, 1)
    issue_a(2, 2)
    wait_a(0, 0)
    issue_adds(0, 0)
    wait_a(1, 1)
    issue_adds(1, 1)

    def body(mm, carry):
        for k in range(4):
            m = 4 * mm + k

            @pl.when(m + 3 < NCHUNK)
            def _(m=m, k=k):
                issue_a(m + 3, (k + 3) % 4)

            @pl.when(m + 2 < NCHUNK)
            def _(m=m, k=k):
                wait_a(m + 2, (k + 2) % 4)
                issue_adds(m + 2, (k + 2) % 4)

            @pl.when(m < NCHUNK)
            def _(m=m, k=k):
                wait_adds(m, k)
                compute(m, k)
        return carry

    lax.fori_loop(0, (NCHUNK + 3) // 4, body, 0)
    pltpu.sync_copy(ov, out_hbm.at[pl.ds(ebase, EPW)])


_BM = 640
_BE = 2000


def _tc_hs0_body(x_ref, w_ref, d0_ref, d1_ref, hs_ref, dinv_ref):
    deg = d0_ref[:, 0:1] + d1_ref[:, 0:1] + 1.0
    dinv = lax.rsqrt(deg)
    h = jnp.dot(x_ref[...], w_ref[...], preferred_element_type=jnp.float32)
    hs_ref[...] = h * dinv
    dinv_ref[...] = dinv


def _tc_hs0(x, w1, degp0, degp1):
    return pl.pallas_call(
        _tc_hs0_body,
        grid=(NP // _BM,),
        in_specs=[
            pl.BlockSpec((_BM, D), lambda i: (i, 0)),
            pl.BlockSpec((D, D), lambda i: (0, 0)),
            pl.BlockSpec((_BM, D), lambda i: (i, 0)),
            pl.BlockSpec((_BM, D), lambda i: (i, 0)),
        ],
        out_specs=[
            pl.BlockSpec((_BM, D), lambda i: (i, 0)),
            pl.BlockSpec((_BM, 1), lambda i: (i, 0)),
        ],
        out_shape=[
            jax.ShapeDtypeStruct((NP, D), jnp.float32),
            jax.ShapeDtypeStruct((NP, 1), jnp.float32),
        ],
    )(x, w1, degp0, degp1)


def _tc_layer_body(a0_ref, a1_ref, hs_ref, dinv_ref, b_ref, w_ref, out_ref):
    dinv = dinv_ref[...]
    h = jnp.maximum(
        (a0_ref[...] + a1_ref[...] + hs_ref[...]) * dinv + b_ref[...], 0.0)
    out_ref[...] = (
        jnp.dot(h, w_ref[...], preferred_element_type=jnp.float32) * dinv)


def _tc_layer(acc0, acc1, hs, dinv, b, w):
    return pl.pallas_call(
        _tc_layer_body,
        grid=(NP // _BM,),
        in_specs=[
            pl.BlockSpec((_BM, D), lambda i: (i, 0)),
            pl.BlockSpec((_BM, D), lambda i: (i, 0)),
            pl.BlockSpec((_BM, D), lambda i: (i, 0)),
            pl.BlockSpec((_BM, 1), lambda i: (i, 0)),
            pl.BlockSpec((1, D), lambda i: (0, 0)),
            pl.BlockSpec((D, D), lambda i: (0, 0)),
        ],
        out_specs=pl.BlockSpec((_BM, D), lambda i: (i, 0)),
        out_shape=jax.ShapeDtypeStruct((NP, D), jnp.float32),
    )(acc0, acc1, hs, dinv, b, w)


def _tc_node_ab_body(a0_ref, a1_ref, hs_ref, dinv_ref, b_ref, wr_ref, wc_ref,
                     aout_ref, bout_ref):
    h = jnp.maximum(
        (a0_ref[...] + a1_ref[...] + hs_ref[...]) * dinv_ref[...] + b_ref[...],
        0.0)
    aout_ref[...] = jnp.dot(h, wr_ref[...], preferred_element_type=jnp.float32)
    bout_ref[...] = jnp.dot(h, wc_ref[...], preferred_element_type=jnp.float32)


def _tc_node_ab(acc0, acc1, hs, dinv, b, wr, wc):
    return pl.pallas_call(
        _tc_node_ab_body,
        grid=(NP // _BM,),
        in_specs=[
            pl.BlockSpec((_BM, D), lambda i: (i, 0)),
            pl.BlockSpec((_BM, D), lambda i: (i, 0)),
            pl.BlockSpec((_BM, D), lambda i: (i, 0)),
            pl.BlockSpec((_BM, 1), lambda i: (i, 0)),
            pl.BlockSpec((1, D), lambda i: (0, 0)),
            pl.BlockSpec((D, D), lambda i: (0, 0)),
            pl.BlockSpec((D, D), lambda i: (0, 0)),
        ],
        out_specs=[
            pl.BlockSpec((_BM, D), lambda i: (i, 0)),
            pl.BlockSpec((_BM, D), lambda i: (i, 0)),
        ],
        out_shape=[
            jax.ShapeDtypeStruct((NP, D), jnp.float32),
            jax.ShapeDtypeStruct((NP, D), jnp.float32),
        ],
    )(acc0, acc1, hs, dinv, b, wr, wc)


def _tc_edge_c_body(ea_ref, w_ref, b_ref, out_ref):
    out_ref[...] = (
        jnp.dot(ea_ref[...], w_ref[...], preferred_element_type=jnp.float32)
        + b_ref[...])


def _tc_edge_c(edge_attr, we, bm1):
    return pl.pallas_call(
        _tc_edge_c_body,
        grid=(E // _BE,),
        in_specs=[
            pl.BlockSpec((_BE, DE), lambda i: (i, 0)),
            pl.BlockSpec((DE, D), lambda i: (0, 0)),
            pl.BlockSpec((1, D), lambda i: (0, 0)),
        ],
        out_specs=pl.BlockSpec((_BE, D), lambda i: (i, 0)),
        out_shape=jax.ShapeDtypeStruct((E, D), jnp.float32),
    )(edge_attr, we, bm1)


def kernel(x, edge_index, edge_attr, W1, b1, W2, b2, Wm1, bm1, Wm2, bm2):
    src2 = edge_index[0].astype(jnp.int32).reshape(NW, EPW)
    src = src2.reshape(NW, NCHUNK, CK)
    dst = edge_index[1].astype(jnp.int32).reshape(NW, NCHUNK, CK)
    xp = jnp.pad(x, ((0, NP - N), (0, 0)))
    zeros128 = jnp.zeros((NP, D), jnp.float32)
    ones128 = jnp.ones((CK, D), jnp.float32)

    degp = _sc_degree(dst, ones128, zeros128)
    hs0, dinv = _tc_hs0(xp, W1, degp[0], degp[1])
    accp1 = _sc_aggregate(hs0, src2, dst, zeros128)
    hs1 = _tc_layer(accp1[0], accp1[1], hs0, dinv, b1.reshape(1, D), W2)
    accp2 = _sc_aggregate(hs1, src2, dst, zeros128)
    a_t, b_t = _tc_node_ab(accp2[0], accp2[1], hs1, dinv, b2.reshape(1, D),
                           Wm1[:D], Wm1[D:2 * D])
    c_t = _tc_edge_c(edge_attr, Wm1[2 * D:], bm1.reshape(1, D))
    logits = _sc_edge_mlp(a_t, b_t, c_t, src, dst, Wm2[:, 0],
                          jnp.full((16,), bm2[0], jnp.float32))
    return logits
